# baseline (speedup 1.0000x reference)
import functools

import jax
import jax.numpy as jnp
from jax import lax
from jax.experimental import pallas as pl
from jax.experimental.pallas import tpu as pltpu
from jax.experimental.pallas import tpu_sc as plsc

_B, _D, _E, _K = 32768, 4096, 64, 8
_BT = 1024
_C = 4
_BC = _B // _C
_NBC = _BC // _BT
_NW = 32
_L = 16
_GPB = _BT // _L
_RPT = _BC // _NW
_NCH = _RPT // _L
_IMIN = jnp.iinfo(jnp.int32).min


def _tc_body(x_ref, w_ref, b_ref, gw_ref, gwb_ref, imp_ref, acc_ref):
    i = pl.program_id(0)
    logits = jnp.dot(x_ref[...], w_ref[...],
                     preferred_element_type=jnp.float32) + b_ref[...]
    lt = logits.T
    m = jnp.max(lt, axis=0, keepdims=True)
    e = jnp.exp(lt - m)
    s = jnp.sum(e, axis=0, keepdims=True)
    gwt = e * (1.0 / s)
    gw_ref[...] = gwt.T
    gwb_ref[...] = jnp.swapaxes(gwt.reshape(_E, _GPB, _L), 0, 1)

    @pl.when(i == 0)
    def _init():
        acc_ref[...] = gwt

    @pl.when(i > 0)
    def _accum():
        acc_ref[...] += gwt

    @pl.when(i == _NBC - 1)
    def _finalize():
        imp_ref[...] = jnp.sum(acc_ref[...], axis=1, keepdims=True).T


def _tc_gate(x_c, W, b2):
    return pl.pallas_call(
        _tc_body,
        grid=(_NBC,),
        in_specs=[
            pl.BlockSpec((_BT, _D), lambda i: (i, 0)),
            pl.BlockSpec((_D, _E), lambda i: (0, 0)),
            pl.BlockSpec((1, _E), lambda i: (0, 0)),
        ],
        out_specs=(
            pl.BlockSpec((_BT, _E), lambda i: (i, 0)),
            pl.BlockSpec((_GPB, _E, _L), lambda i: (i, 0, 0)),
            pl.BlockSpec((1, _E), lambda i: (0, 0)),
        ),
        out_shape=(
            jax.ShapeDtypeStruct((_BC, _E), jnp.float32),
            jax.ShapeDtypeStruct((_BC // _L, _E, _L), jnp.float32),
            jax.ShapeDtypeStruct((1, _E), jnp.float32),
        ),
        scratch_shapes=[pltpu.VMEM((_E, _BT), jnp.float32)],
    )(x_c, W, b2)


_SC_MESH = plsc.VectorSubcoreMesh(core_axis_name="c", subcore_axis_name="s")


@functools.partial(
    pl.kernel,
    mesh=_SC_MESH,
    out_type=[
        jax.ShapeDtypeStruct((_NW * _NCH * _K * _L,), jnp.int32),
        jax.ShapeDtypeStruct((_NW * _NCH * _K * _L,), jnp.float32),
        jax.ShapeDtypeStruct((_NW, _E * _L), jnp.float32),
        jax.ShapeDtypeStruct((_NW, _L), jnp.float32),
    ],
    scratch_types=[
        pltpu.VMEM((_E * _L,), jnp.float32),
        pltpu.VMEM((_E * _L,), jnp.float32),
        pltpu.VMEM((_K * _L,), jnp.int32),
        pltpu.VMEM((_K * _L,), jnp.float32),
        pltpu.VMEM((_L,), jnp.float32),
    ],
)
def _sc_route(gwb_hbm, tki_hbm, tkw_hbm, cnt_hbm, tks_hbm,
              in_v, cnt_v, oi_v, ow_v, tks_v):
    wid = lax.axis_index("s") * 2 + lax.axis_index("c")
    zeros16 = jnp.zeros((_L,), jnp.float32)
    neg16 = jnp.full((_L,), -1.0, jnp.float32)
    none16 = jnp.full((_L,), -1, jnp.int32)

    for e in range(_E):
        cnt_v[pl.ds(e * _L, _L)] = zeros16

    def body(ci, tks_acc):
        g = wid * _NCH + ci
        pltpu.sync_copy(gwb_hbm.at[pl.ds(g * _E * _L, _E * _L)], in_v)
        prev_mi = none16
        for k in range(_K):
            m = neg16
            mi = none16
            for e in range(_E):
                kv = in_v[pl.ds(e * _L, _L)]
                kv = jnp.where(prev_mi == e, -1.0, kv)
                in_v[pl.ds(e * _L, _L)] = kv
                cond = kv > m
                m = jnp.where(cond, kv, m)
                mi = jnp.where(cond, e, mi)
            oi_v[pl.ds(k * _L, _L)] = mi
            ow_v[pl.ds(k * _L, _L)] = m
            tks_acc = tks_acc + m
            prev_mi = mi
        for e in range(_E):
            kv = in_v[pl.ds(e * _L, _L)]
            hit = (kv < 0.0) | (prev_mi == e)
            cnt_v[pl.ds(e * _L, _L)] += jnp.where(hit, 1.0, 0.0)
        obase = (wid * _NCH + ci) * _K * _L
        pltpu.sync_copy(oi_v, tki_hbm.at[pl.ds(obase, _K * _L)])
        pltpu.sync_copy(ow_v, tkw_hbm.at[pl.ds(obase, _K * _L)])
        return tks_acc

    tks = lax.fori_loop(0, _NCH, body, zeros16)
    tks_v[...] = tks
    pltpu.sync_copy(cnt_v, cnt_hbm.at[wid])
    pltpu.sync_copy(tks_v, tks_hbm.at[wid])


@functools.partial(jax.jit, static_argnames=())
def kernel(x, W, b):
    b2 = b.reshape(1, _E)
    gw_l, tki_l, tkw_l, cnt_l, tks_l = [], [], [], [], []
    imp_sum = jnp.zeros((1, _E), jnp.float32)
    for c in range(_C):
        x_c = lax.slice_in_dim(x, c * _BC, (c + 1) * _BC, axis=0)
        gw_c, gwb_c, imp_c = _tc_gate(x_c, W, b2)
        tki_c, tkw_c, cnt_c, tks_c = _sc_route(gwb_c.reshape(-1))
        tki_l.append(
            tki_c.reshape(_NW * _NCH, _K, _L).swapaxes(1, 2).reshape(_BC, _K))
        tkw_l.append(
            tkw_c.reshape(_NW * _NCH, _K, _L).swapaxes(1, 2).reshape(_BC, _K))
        gw_l.append(gw_c)
        cnt_l.append(cnt_c)
        tks_l.append(tks_c)
        imp_sum = imp_sum + imp_c
    gw = jnp.concatenate(gw_l, axis=0)
    tki = jnp.concatenate(tki_l, axis=0)
    tkw = jnp.concatenate(tkw_l, axis=0)
    counts = sum(
        c.reshape(_NW, _E, _L).sum(axis=(0, 2)) for c in cnt_l)
    importance = imp_sum.reshape(_E) * (1.0 / _B)
    load = counts * (1.0 / _B)
    lb = _E * jnp.sum(importance * load)
    sp = 1.0 - sum(jnp.sum(t) for t in tks_l) * (1.0 / _B)
    return (gw, tki, tkw, lb, sp)


# speedup vs baseline: 3.1596x; 3.1596x over previous
import functools

import jax
import jax.numpy as jnp
from jax import lax
from jax.experimental import pallas as pl
from jax.experimental.pallas import tpu as pltpu

_B, _D, _E, _K = 32768, 4096, 64, 8
_BT = 1024
_NB = _B // _BT


def _router_body(x_ref, w_ref, b_ref, gw_ref, tki_ref, tkw_ref, lb_ref,
                 sp_ref, lt_ref, imp_ref, cnt_ref, tks_ref):
    i = pl.program_id(0)

    logits = jnp.dot(x_ref[...], w_ref[...],
                     preferred_element_type=jnp.float32) + b_ref[...]
    lt_ref[i % 2] = logits.T

    lt = lt_ref[(i + 1) % 2]
    m = jnp.max(lt, axis=0, keepdims=True)
    e = jnp.exp(lt - m)
    s = jnp.sum(e, axis=0, keepdims=True)
    gwt = e * (1.0 / s)
    gw_ref[...] = gwt.T

    iota = lax.broadcasted_iota(jnp.int32, (_E, _BT), 0)
    g = gwt
    w_rows = []
    i_rows = []
    for _ in range(_K):
        mx = jnp.max(g, axis=0, keepdims=True)
        idx = jnp.min(jnp.where(g == mx, iota, _E), axis=0, keepdims=True)
        w_rows.append(mx)
        i_rows.append(idx)
        g = jnp.where(iota == idx, -1.0, g)
    tkw_t = jnp.concatenate(w_rows, axis=0)
    tki_t = jnp.concatenate(i_rows, axis=0)
    tkw_ref[...] = tkw_t.T
    tki_ref[...] = tki_t.T

    sel = (g < 0.0).astype(jnp.float32)
    tks_blk = jnp.sum(tkw_t, axis=0, keepdims=True)

    fresh = i <= 1
    imp_ref[...] = jnp.where(fresh, 0.0, imp_ref[...]) + gwt
    cnt_ref[...] = jnp.where(fresh, 0.0, cnt_ref[...]) + sel
    tks_ref[...] = jnp.where(fresh, 0.0, tks_ref[...]) + tks_blk

    @pl.when(i == _NB)
    def _finalize():
        imp_col = jnp.sum(imp_ref[...], axis=1, keepdims=True)
        cnt_col = jnp.sum(cnt_ref[...], axis=1, keepdims=True)
        lb_ref[0, 0] = (_E / (_B * float(_B))) * jnp.sum(imp_col * cnt_col)
        sp_ref[0, 0] = 1.0 - jnp.sum(tks_ref[...]) * (1.0 / _B)


@functools.partial(jax.jit, static_argnames=())
def kernel(x, W, b):
    b2 = b.reshape(1, _E)
    out_shape = (
        jax.ShapeDtypeStruct((_B, _E), jnp.float32),
        jax.ShapeDtypeStruct((_B, _K), jnp.int32),
        jax.ShapeDtypeStruct((_B, _K), jnp.float32),
        jax.ShapeDtypeStruct((1, 1), jnp.float32),
        jax.ShapeDtypeStruct((1, 1), jnp.float32),
    )
    grid = (_NB + 1,)
    prev = lambda i: (jnp.maximum(i - 1, 0), 0)
    gw, tki, tkw, lb, sp = pl.pallas_call(
        _router_body,
        grid=grid,
        in_specs=[
            pl.BlockSpec((_BT, _D), lambda i: (jnp.minimum(i, _NB - 1), 0)),
            pl.BlockSpec((_D, _E), lambda i: (0, 0)),
            pl.BlockSpec((1, _E), lambda i: (0, 0)),
        ],
        out_specs=(
            pl.BlockSpec((_BT, _E), prev),
            pl.BlockSpec((_BT, _K), prev),
            pl.BlockSpec((_BT, _K), prev),
            pl.BlockSpec(memory_space=pltpu.SMEM),
            pl.BlockSpec(memory_space=pltpu.SMEM),
        ),
        out_shape=out_shape,
        scratch_shapes=[
            pltpu.VMEM((2, _E, _BT), jnp.float32),
            pltpu.VMEM((_E, _BT), jnp.float32),
            pltpu.VMEM((_E, _BT), jnp.float32),
            pltpu.VMEM((1, _BT), jnp.float32),
        ],
    )(x, W, b2)
    return (gw, tki, tkw, lb.reshape(()), sp.reshape(()))


# speedup vs baseline: 3.2024x; 1.0135x over previous
import functools

import jax
import jax.numpy as jnp
from jax import lax
from jax.experimental import pallas as pl
from jax.experimental.pallas import tpu as pltpu

_B, _D, _E, _K = 32768, 4096, 64, 8
_BT = 1024
_NB = _B // _BT


def _router_body(x_ref, w_ref, b_ref, gw_ref, tki_ref, tkw_ref, lb_ref,
                 sp_ref, imp_ref, cnt_ref, tks_ref):
    i = pl.program_id(0)

    logits = jnp.dot(x_ref[...], w_ref[...],
                     preferred_element_type=jnp.float32) + b_ref[...]
    lt = logits.T
    m = jnp.max(lt, axis=0, keepdims=True)
    e = jnp.exp(lt - m)
    s = jnp.sum(e, axis=0, keepdims=True)
    gwt = e * (1.0 / s)
    gw_ref[...] = gwt.T

    iota = lax.broadcasted_iota(jnp.int32, (_E, _BT), 0)
    g = gwt
    w_rows = []
    i_rows = []
    for _ in range(_K):
        mx = jnp.max(g, axis=0, keepdims=True)
        idx = jnp.min(jnp.where(g == mx, iota, _E), axis=0, keepdims=True)
        w_rows.append(mx)
        i_rows.append(idx)
        g = jnp.where(iota == idx, -1.0, g)
    tkw_t = jnp.concatenate(w_rows, axis=0)
    tki_t = jnp.concatenate(i_rows, axis=0)
    tkw_ref[...] = tkw_t.T
    tki_ref[...] = tki_t.T

    sel = (g < 0.0).astype(jnp.float32)

    @pl.when(i == 0)
    def _init():
        imp_ref[...] = gwt
        cnt_ref[...] = sel
        tks_ref[...] = jnp.sum(tkw_t, axis=0, keepdims=True)

    @pl.when(i > 0)
    def _accum():
        imp_ref[...] += gwt
        cnt_ref[...] += sel
        tks_ref[...] += jnp.sum(tkw_t, axis=0, keepdims=True)

    @pl.when(i == _NB - 1)
    def _finalize():
        imp_col = jnp.sum(imp_ref[...], axis=1, keepdims=True)
        cnt_col = jnp.sum(cnt_ref[...], axis=1, keepdims=True)
        lb_ref[0, 0] = (_E / (_B * float(_B))) * jnp.sum(imp_col * cnt_col)
        sp_ref[0, 0] = 1.0 - jnp.sum(tks_ref[...]) * (1.0 / _B)


@functools.partial(jax.jit, static_argnames=())
def kernel(x, W, b):
    b2 = b.reshape(1, _E)
    out_shape = (
        jax.ShapeDtypeStruct((_B, _E), jnp.float32),
        jax.ShapeDtypeStruct((_B, _K), jnp.int32),
        jax.ShapeDtypeStruct((_B, _K), jnp.float32),
        jax.ShapeDtypeStruct((1, 1), jnp.float32),
        jax.ShapeDtypeStruct((1, 1), jnp.float32),
    )
    grid = (_NB,)
    gw, tki, tkw, lb, sp = pl.pallas_call(
        _router_body,
        grid=grid,
        in_specs=[
            pl.BlockSpec((_BT, _D), lambda i: (i, 0)),
            pl.BlockSpec((_D, _E), lambda i: (0, 0)),
            pl.BlockSpec((1, _E), lambda i: (0, 0)),
        ],
        out_specs=(
            pl.BlockSpec((_BT, _E), lambda i: (i, 0)),
            pl.BlockSpec((_BT, _K), lambda i: (i, 0)),
            pl.BlockSpec((_BT, _K), lambda i: (i, 0)),
            pl.BlockSpec(memory_space=pltpu.SMEM),
            pl.BlockSpec(memory_space=pltpu.SMEM),
        ),
        out_shape=out_shape,
        scratch_shapes=[
            pltpu.VMEM((_E, _BT), jnp.float32),
            pltpu.VMEM((_E, _BT), jnp.float32),
            pltpu.VMEM((1, _BT), jnp.float32),
        ],
    )(x, W, b2)
    return (gw, tki, tkw, lb.reshape(()), sp.reshape(()))


# speedup vs baseline: 3.2029x; 1.0002x over previous
import functools

import jax
import jax.numpy as jnp
from jax import lax
from jax.experimental import pallas as pl
from jax.experimental.pallas import tpu as pltpu

_B, _D, _E, _K = 32768, 4096, 64, 8
_BT = 1024
_NB = _B // _BT


def _router_body(x_ref, w_ref, b_ref, gw_ref, tki_ref, tkw_ref, lb_ref,
                 sp_ref, imp_ref, cnt_ref, tks_ref):
    i = pl.program_id(0)

    logits = jnp.dot(x_ref[...], w_ref[...],
                     preferred_element_type=jnp.float32) + b_ref[...]
    lt = logits.T
    m = jnp.max(lt, axis=0, keepdims=True)
    e = jnp.exp(lt - m)
    s = jnp.sum(e, axis=0, keepdims=True)
    gwt = e / s
    gw_ref[...] = gwt.T

    iota = lax.broadcasted_iota(jnp.int32, (_E, _BT), 0)
    g = gwt
    w_rows = []
    i_rows = []
    for _ in range(_K):
        mx = jnp.max(g, axis=0, keepdims=True)
        idx = jnp.min(jnp.where(g == mx, iota, _E), axis=0, keepdims=True)
        w_rows.append(mx)
        i_rows.append(idx)
        g = jnp.where(iota == idx, -1.0, g)
    tkw_t = jnp.concatenate(w_rows, axis=0)
    tki_t = jnp.concatenate(i_rows, axis=0)
    tkw_ref[...] = tkw_t.T
    tki_ref[...] = tki_t.T

    sel = (g < 0.0).astype(jnp.float32)

    @pl.when(i == 0)
    def _init():
        imp_ref[...] = gwt
        cnt_ref[...] = sel
        tks_ref[...] = jnp.sum(tkw_t, axis=0, keepdims=True)

    @pl.when(i > 0)
    def _accum():
        imp_ref[...] += gwt
        cnt_ref[...] += sel
        tks_ref[...] += jnp.sum(tkw_t, axis=0, keepdims=True)

    @pl.when(i == _NB - 1)
    def _finalize():
        imp_col = jnp.sum(imp_ref[...], axis=1, keepdims=True)
        cnt_col = jnp.sum(cnt_ref[...], axis=1, keepdims=True)
        lb_ref[0, 0] = (_E / (_B * float(_B))) * jnp.sum(imp_col * cnt_col)
        sp_ref[0, 0] = 1.0 - jnp.sum(tks_ref[...]) * (1.0 / _B)


@functools.partial(jax.jit, static_argnames=())
def kernel(x, W, b):
    b2 = b.reshape(1, _E)
    out_shape = (
        jax.ShapeDtypeStruct((_B, _E), jnp.float32),
        jax.ShapeDtypeStruct((_B, _K), jnp.int32),
        jax.ShapeDtypeStruct((_B, _K), jnp.float32),
        jax.ShapeDtypeStruct((1, 1), jnp.float32),
        jax.ShapeDtypeStruct((1, 1), jnp.float32),
    )
    grid = (_NB,)
    gw, tki, tkw, lb, sp = pl.pallas_call(
        _router_body,
        grid=grid,
        in_specs=[
            pl.BlockSpec((_BT, _D), lambda i: (i, 0)),
            pl.BlockSpec((_D, _E), lambda i: (0, 0)),
            pl.BlockSpec((1, _E), lambda i: (0, 0)),
        ],
        out_specs=(
            pl.BlockSpec((_BT, _E), lambda i: (i, 0)),
            pl.BlockSpec((_BT, _K), lambda i: (i, 0)),
            pl.BlockSpec((_BT, _K), lambda i: (i, 0)),
            pl.BlockSpec(memory_space=pltpu.SMEM),
            pl.BlockSpec(memory_space=pltpu.SMEM),
        ),
        out_shape=out_shape,
        scratch_shapes=[
            pltpu.VMEM((_E, _BT), jnp.float32),
            pltpu.VMEM((_E, _BT), jnp.float32),
            pltpu.VMEM((1, _BT), jnp.float32),
        ],
    )(x, W, b2)
    return (gw, tki, tkw, lb.reshape(()), sp.reshape(()))
